# baseline (speedup 1.0000x reference)
import functools
import math

import jax
import jax.numpy as jnp
from jax import lax
from jax.experimental import pallas as pl
from jax.experimental.pallas import tpu as pltpu
from jax.experimental.pallas import tpu_sc as plsc

_DELTA_VAR = 0.5
_PMAPS_THRESHOLD = 0.9
_OVERLAP_THRESHOLD = 0.5
_TWO_SIGMA = _DELTA_VAR * _DELTA_VAR / -math.log(_PMAPS_THRESHOLD)
_D2_THRESH = _TWO_SIGMA * math.log(2.0)
_NUM_IDS = 20
_G = _NUM_IDS - 1

_LANES = 128
_ABSENT = 1.0e12


def _sc_anchor_body(b, c, hw, targ_hbm, pred_hbm, anch_hbm,
                    t_v, row_v, tmp_v, idxtab_v, prestab_v, idxv, rows_v,
                    shared, sem):
  nc, ns, lanes = 2, 16, 16
  nw = nc * ns
  chunk = hw // nw
  wid = lax.axis_index("s") * nc + lax.axis_index("c")
  lane16 = lax.broadcasted_iota(jnp.int32, (lanes,), 0)
  nvec = chunk // lanes
  cg = c * _G

  for bi in range(b):
    pltpu.sync_copy(targ_hbm.at[pl.ds(bi * hw + wid * chunk, chunk)], t_v)
    base = wid * chunk

    def scan_body(v, best):
      tv = t_v[pl.ds(v * lanes, lanes)]
      iv = base + v * lanes + lane16
      out = []
      for g in range(1, _NUM_IDS):
        out.append(jnp.minimum(best[g - 1], jnp.where(tv == g, iv, hw)))
      return tuple(out)

    init = tuple(jnp.full((lanes,), hw, jnp.int32) for _ in range(_G))
    best = lax.fori_loop(0, nvec, scan_body, init)

    for g in range(1, _NUM_IDS):
      row_v[pl.ds((g - 1) * lanes, lanes)] = best[g - 1]
    row_v[pl.ds(_G * lanes, lanes)] = jnp.full((lanes,), hw, jnp.int32)
    pltpu.sync_copy(row_v, shared.at[bi, wid])

  plsc.subcore_barrier()

  @pl.when(wid < b)
  def _finalize():
    bi = wid
    acc = [jnp.full((lanes,), hw, jnp.int32) for _ in range(_G)]
    for w2 in range(nw):
      pltpu.sync_copy(shared.at[bi, w2], tmp_v)
      for g in range(1, _NUM_IDS):
        acc[g - 1] = jnp.minimum(acc[g - 1],
                                 tmp_v[pl.ds((g - 1) * lanes, lanes)])
    idx_s = []
    pres_f = []
    for g in range(1, _NUM_IDS):
      v = acc[g - 1]
      mn = v[0]
      for k in range(1, lanes):
        mn = jnp.minimum(mn, v[k])
      idx_s.append(jnp.minimum(mn, hw - 1))
      pres_f.append(jnp.where(mn < hw, 1.0, 0.0))

    nvec_cg = (cg + lanes - 1) // lanes
    for j in range(nvec_cg):
      gj = j >> 1
      cc = (j & 1) * lanes + lane16
      idxv[pl.ds(j * lanes, lanes)] = (bi * c + cc) * hw + idx_s[gj]
    for j in range(nvec_cg, idxv.shape[0] // lanes):
      idxv[pl.ds(j * lanes, lanes)] = jnp.zeros((lanes,), jnp.int32)

    npad = idxv.shape[0]
    copies = []
    for j2 in range(npad // _LANES):
      cp = pltpu.make_async_copy(
          pred_hbm.at[idxv.at[pl.ds(j2 * _LANES, _LANES)]],
          rows_v.at[pl.ds(j2 * _LANES, _LANES)],
          sem)
      cp.start()
      copies.append(cp)
    for cp in copies:
      cp.wait()

    for j in range(nvec_cg):
      gj = j >> 1
      m = pres_f[gj]
      rv = rows_v[pl.ds(j * lanes, lanes)]
      rows_v[pl.ds(j * lanes, lanes)] = (m * (-2.0 * rv)
                                         + (1.0 - m) * _ABSENT)

    pltpu.sync_copy(rows_v, anch_hbm.at[bi])


def _dgt(lhs, rhs, dims):
  return lax.dot_general(lhs, rhs, (dims, ((), ())),
                         preferred_element_type=jnp.float32)


def _main_body(nb, targ_ref, pred_ref, anch_ref,
               score_ref, cnt_ts, cnt_ss, anorm_s):
  i = pl.program_id(1)

  p = pred_ref[0]
  a2 = anch_ref[0]
  t = targ_ref[0]
  c, blk = p.shape

  @pl.when(i == 0)
  def _init():
    cnt_ts[...] = jnp.zeros_like(cnt_ts)
    cnt_ss[...] = jnp.zeros_like(cnt_ss)
    anorm_s[...] = (jnp.sum(a2 * a2, axis=0) * 0.25).reshape(_G, 1)

  anorm = anorm_s[...]

  dot = _dgt(a2, p, ((0,), (0,)))
  pnorm = _dgt(jnp.ones((1, c), jnp.float32), p * p, ((1,), (0,)))
  s = dot + anorm
  rhs = _D2_THRESH - pnorm
  condf = (s < rhs).astype(jnp.float32)

  w = (1 << (lax.broadcasted_iota(jnp.int32, (1, _G), 1) + 1)
       ).astype(jnp.float32)
  u = _dgt(w, condf, ((1,), (0,)))
  ubits = lax.bitcast_convert_type(u + 1.0, jnp.int32)
  seg = (ubits >> 23) - 127

  gid = lax.broadcasted_iota(jnp.int32, (_G, blk), 0) + 1
  tmf = (t == gid).astype(jnp.float32)
  smf = (seg == gid).astype(jnp.float32)
  e = ((seg == t) & (t > 0)).astype(jnp.float32)
  ones_row = jnp.ones((1, blk), jnp.float32)
  rhs2 = jnp.concatenate([e, ones_row], axis=0)
  cnt_ts[...] += _dgt(tmf, rhs2, ((1,), (1,)))
  cnt_ss[...] += _dgt(smf, rhs2, ((1,), (1,)))

  @pl.when(i == nb - 1)
  def _fin():
    inter = cnt_ts[:, 0:1]
    nt = cnt_ts[:, 1:2]
    nseg = cnt_ss[:, 1:2]
    union = nseg + nt - inter
    iou = inter / jnp.maximum(union, 1.0)
    gtp = (nt > 0.0).astype(jnp.float32)
    prp = (nseg > 0.0).astype(jnp.float32)
    matched = (iou > _OVERLAP_THRESHOLD).astype(jnp.float32) * gtp * prp
    tp = jnp.sum(matched)
    denom = jnp.sum(gtp) + jnp.sum(prp) - tp
    score = tp / jnp.maximum(denom, 1.0)
    score_ref[...] = jnp.reshape(score, (1, 1, 1))


def kernel(pred, target):
  b, c, h, w = pred.shape
  hw = h * w
  blk = 65536
  nb = hw // blk
  cg = c * _G
  cg_pad = ((cg + _LANES - 1) // _LANES) * _LANES

  pred3 = pred.reshape(b, c, hw)
  predflat = pred.reshape(b * c * hw)
  targflat = target.reshape(b * hw)
  targ3 = target.reshape(b, 1, hw)

  mesh = plsc.VectorSubcoreMesh(core_axis_name="c", subcore_axis_name="s")
  sc_anchor = functools.partial(
      pl.kernel,
      mesh=mesh,
      out_type=jax.ShapeDtypeStruct((b, cg_pad), jnp.float32),
      scratch_types=[
          pltpu.VMEM((hw // 32,), jnp.int32),
          pltpu.VMEM((20 * 16,), jnp.int32),
          pltpu.VMEM((20 * 16,), jnp.int32),
          pltpu.VMEM((32,), jnp.int32),
          pltpu.VMEM((32,), jnp.float32),
          pltpu.VMEM((cg_pad,), jnp.int32),
          pltpu.VMEM((cg_pad,), jnp.float32),
          pltpu.VMEM_SHARED((b, 32, 20 * 16), jnp.int32),
          pltpu.SemaphoreType.DMA,
      ],
  )(functools.partial(_sc_anchor_body, b, c, hw))
  anch2 = sc_anchor(targflat, predflat)
  anchors = anch2[:, :cg].reshape(b, _G, c).transpose(0, 2, 1)

  score = pl.pallas_call(
      functools.partial(_main_body, nb),
      grid=(b, nb),
      in_specs=[
          pl.BlockSpec((1, 1, blk), lambda bi, i: (bi, 0, i)),
          pl.BlockSpec((1, c, blk), lambda bi, i: (bi, 0, i)),
          pl.BlockSpec((1, c, _G), lambda bi, i: (bi, 0, 0)),
      ],
      out_specs=pl.BlockSpec((1, 1, 1), lambda bi, i: (bi, 0, 0)),
      out_shape=jax.ShapeDtypeStruct((b, 1, 1), jnp.float32),
      scratch_shapes=[
          pltpu.VMEM((_G, 2), jnp.float32),
          pltpu.VMEM((_G, 2), jnp.float32),
          pltpu.VMEM((_G, 1), jnp.float32),
      ],
  )(targ3, pred3, anchors)

  return score.reshape(b)
